# baseline (speedup 1.0000x reference)
import functools
import jax
import jax.numpy as jnp
from jax import lax
from jax.experimental import pallas as pl
from jax.experimental.pallas import tpu as pltpu

NP = 10240
EP = 163840
EB = 2048
NB = 256
G = 256

_lrelu = lambda x: jnp.where(x >= 0, x, 0.01 * x)


def _elu(x):
    return jnp.where(x > 0, x, jnp.exp(jnp.minimum(x, 0.0)) - 1.0)


def _gru(x, h, Wih, bih, Whh, bhh):
    gi = jnp.dot(x, Wih, preferred_element_type=jnp.float32) + bih
    gh = jnp.dot(h, Whh, preferred_element_type=jnp.float32) + bhh
    i_r, i_z, i_n = gi[:, :G], gi[:, G:2 * G], gi[:, 2 * G:]
    h_r, h_z, h_n = gh[:, :G], gh[:, G:2 * G], gh[:, 2 * G:]
    r = jax.nn.sigmoid(i_r + h_r)
    z = jax.nn.sigmoid(i_z + h_z)
    nw = jnp.tanh(i_n + r * h_n)
    return (1.0 - z) * nw + z * h


def _k_prep(nf, Wn, bn, We1a, be1, we2d, be2, hv, u, d1):
    t1 = jnp.dot(nf[...], Wn[...], preferred_element_type=jnp.float32) + bn[...]
    hvv = _lrelu(t1)
    hv[...] = hvv
    u[...] = jnp.dot(nf[...], We1a[...], preferred_element_type=jnp.float32) + be1[...]
    d1[...] = jnp.dot(hvv, we2d[...], preferred_element_type=jnp.float32) + be2[...]


def tc_prep(nf_p, Wn, bn, We1a, be1, we2d, be2):
    n = NP // NB
    return pl.pallas_call(
        _k_prep,
        grid=(n,),
        in_specs=[
            pl.BlockSpec((NB, 256), lambda i: (i, 0)),
            pl.BlockSpec((256, 256), lambda i: (0, 0)),
            pl.BlockSpec((1, 256), lambda i: (0, 0)),
            pl.BlockSpec((256, 256), lambda i: (0, 0)),
            pl.BlockSpec((1, 256), lambda i: (0, 0)),
            pl.BlockSpec((256, 1), lambda i: (0, 0)),
            pl.BlockSpec((1, 1), lambda i: (0, 0)),
        ],
        out_specs=[
            pl.BlockSpec((NB, 256), lambda i: (i, 0)),
            pl.BlockSpec((NB, 256), lambda i: (i, 0)),
            pl.BlockSpec((NB, 1), lambda i: (i, 0)),
        ],
        out_shape=[
            jax.ShapeDtypeStruct((NP, 256), jnp.float32),
            jax.ShapeDtypeStruct((NP, 256), jnp.float32),
            jax.ShapeDtypeStruct((NP, 1), jnp.float32),
        ],
    )(nf_p, Wn, bn.reshape(1, 256), We1a, be1.reshape(1, 256), we2d, be2.reshape(1, 1))


def _k_passA(g, ef, dvec, We1b, we, ehh, e1):
    v = jnp.dot(ef[...], We1b[...], preferred_element_type=jnp.float32)
    he1 = _lrelu(g[...] + v)
    l = _lrelu(jnp.dot(he1, we[...], preferred_element_type=jnp.float32) + dvec[...])
    e = jnp.exp(jnp.minimum(l, 45.0))
    eh = e * he1
    ehh[0] = eh[:, :128]
    ehh[1] = eh[:, 128:]
    e1[...] = e


def tc_passA(g, ef_p, dvec, We1b, we):
    n = EP // EB
    return pl.pallas_call(
        _k_passA,
        grid=(n,),
        in_specs=[
            pl.BlockSpec((EB, 256), lambda i: (i, 0)),
            pl.BlockSpec((EB, 16), lambda i: (i, 0)),
            pl.BlockSpec((EB, 1), lambda i: (i, 0)),
            pl.BlockSpec((16, 256), lambda i: (0, 0)),
            pl.BlockSpec((256, 1), lambda i: (0, 0)),
        ],
        out_specs=[
            pl.BlockSpec((2, EB, 128), lambda i: (0, i, 0)),
            pl.BlockSpec((EB, 1), lambda i: (i, 0)),
        ],
        out_shape=[
            jax.ShapeDtypeStruct((2, EP, 128), jnp.float32),
            jax.ShapeDtypeStruct((EP, 1), jnp.float32),
        ],
    )(g, ef_p, dvec, We1b, we)


def _k_passB(g2, e1, ehh):
    eh = e1[...] * g2[...]
    ehh[0] = eh[:, :128]
    ehh[1] = eh[:, 128:]


def tc_passB(g2, e1):
    n = EP // EB
    return pl.pallas_call(
        _k_passB,
        grid=(n,),
        in_specs=[
            pl.BlockSpec((EB, 256), lambda i: (i, 0)),
            pl.BlockSpec((EB, 1), lambda i: (i, 0)),
        ],
        out_specs=pl.BlockSpec((2, EB, 128), lambda i: (0, i, 0)),
        out_shape=jax.ShapeDtypeStruct((2, EP, 128), jnp.float32),
    )(g2, e1)


def _k_ctx_gru(Pa, Pb, s0, s1, hv, Wet, bet, Wih, bih, Whh, bhh, hout, *, use_wet):
    P = jnp.concatenate([Pa[0] + Pa[1], Pb[0] + Pb[1]], axis=1)
    s = s0[...] + s1[...]
    Pn = P / jnp.maximum(s, 1e-30)
    if use_wet:
        c = jnp.dot(Pn, Wet[...], preferred_element_type=jnp.float32) \
            + jnp.where(s > 0, 1.0, 0.0) * bet[...]
    else:
        c = Pn
    h = _gru(_elu(c), hv[...], Wih[...], bih[...], Whh[...], bhh[...])
    hout[...] = jnp.maximum(h, 0.0)


def tc_ctx_gru(Pa, Pb, s, hv, Wet, bet, Wih, bih, Whh, bhh, use_wet):
    n = NP // NB
    s0 = s[0].reshape(NP, 1)
    s1 = s[1].reshape(NP, 1)
    return pl.pallas_call(
        functools.partial(_k_ctx_gru, use_wet=use_wet),
        grid=(n,),
        in_specs=[
            pl.BlockSpec((2, NB, 128), lambda i: (0, i, 0)),
            pl.BlockSpec((2, NB, 128), lambda i: (0, i, 0)),
            pl.BlockSpec((NB, 1), lambda i: (i, 0)),
            pl.BlockSpec((NB, 1), lambda i: (i, 0)),
            pl.BlockSpec((NB, 256), lambda i: (i, 0)),
            pl.BlockSpec((256, 256), lambda i: (0, 0)),
            pl.BlockSpec((1, 256), lambda i: (0, 0)),
            pl.BlockSpec((256, 768), lambda i: (0, 0)),
            pl.BlockSpec((1, 768), lambda i: (0, 0)),
            pl.BlockSpec((256, 768), lambda i: (0, 0)),
            pl.BlockSpec((1, 768), lambda i: (0, 0)),
        ],
        out_specs=pl.BlockSpec((NB, 256), lambda i: (i, 0)),
        out_shape=jax.ShapeDtypeStruct((NP, 256), jnp.float32),
    )(Pa, Pb, s0, s1, hv, Wet, bet.reshape(1, 256), Wih, bih.reshape(1, 768),
      Whh, bhh.reshape(1, 768))


def _k_proj(h, Wpn, bpn, wd, ws, hvp, wdt, wst):
    hh = h[...]
    hvp[...] = jnp.dot(hh, Wpn[...], preferred_element_type=jnp.float32) + bpn[...]
    wdt[...] = jnp.dot(hh, wd[...], preferred_element_type=jnp.float32)
    wst[...] = jnp.dot(hh, ws[...], preferred_element_type=jnp.float32)


def tc_proj(h, Wpn, bpn, wd, ws):
    n = NP // NB
    return pl.pallas_call(
        _k_proj,
        grid=(n,),
        in_specs=[
            pl.BlockSpec((NB, 256), lambda i: (i, 0)),
            pl.BlockSpec((256, 256), lambda i: (0, 0)),
            pl.BlockSpec((1, 256), lambda i: (0, 0)),
            pl.BlockSpec((256, 1), lambda i: (0, 0)),
            pl.BlockSpec((256, 1), lambda i: (0, 0)),
        ],
        out_specs=[
            pl.BlockSpec((NB, 256), lambda i: (i, 0)),
            pl.BlockSpec((NB, 1), lambda i: (i, 0)),
            pl.BlockSpec((NB, 1), lambda i: (i, 0)),
        ],
        out_shape=[
            jax.ShapeDtypeStruct((NP, 256), jnp.float32),
            jax.ShapeDtypeStruct((NP, 1), jnp.float32),
            jax.ShapeDtypeStruct((NP, 1), jnp.float32),
        ],
    )(h, Wpn, bpn.reshape(1, 256), wd, ws)


def _k_pred1(h, W1, b1, x, acc):
    i = pl.program_id(0)
    xv = jnp.maximum(jnp.dot(h[...], W1[...], preferred_element_type=jnp.float32)
                     + b1[...], 0.0)
    x[...] = xv
    row = i * NB + lax.broadcasted_iota(jnp.int32, (NB, 1), 0)
    m = jnp.where(row < 10000, 1.0, 0.0)
    xm = xv * m
    part = jnp.concatenate([jnp.sum(xm, axis=0, keepdims=True),
                            jnp.sum(xm * xm, axis=0, keepdims=True),
                            jnp.zeros((6, 256), jnp.float32)], axis=0)

    @pl.when(i == 0)
    def _():
        acc[...] = jnp.zeros_like(acc)
    acc[...] += part


def tc_pred1(h, W1, b1):
    n = NP // NB
    return pl.pallas_call(
        _k_pred1,
        grid=(n,),
        in_specs=[
            pl.BlockSpec((NB, 256), lambda i: (i, 0)),
            pl.BlockSpec((256, 256), lambda i: (0, 0)),
            pl.BlockSpec((1, 256), lambda i: (0, 0)),
        ],
        out_specs=[
            pl.BlockSpec((NB, 256), lambda i: (i, 0)),
            pl.BlockSpec((8, 256), lambda i: (0, 0)),
        ],
        out_shape=[
            jax.ShapeDtypeStruct((NP, 256), jnp.float32),
            jax.ShapeDtypeStruct((8, 256), jnp.float32),
        ],
    )(h, W1, b1.reshape(1, 256))


def _k_pred2(x, acc, gamma, beta, W2, b2, out):
    cnt = 10000.0
    mu = acc[0:1, :] / cnt
    var = acc[1:2, :] / cnt - mu * mu
    inv = gamma[...] / jnp.sqrt(var + 1e-5)
    xn = (x[...] - mu) * inv + beta[...]
    out[...] = jnp.dot(xn, W2[...], preferred_element_type=jnp.float32) + b2[...]


def tc_pred2(x, acc, gamma, beta, W2, b2):
    n = NP // NB
    return pl.pallas_call(
        _k_pred2,
        grid=(n,),
        in_specs=[
            pl.BlockSpec((NB, 256), lambda i: (i, 0)),
            pl.BlockSpec((8, 256), lambda i: (0, 0)),
            pl.BlockSpec((1, 256), lambda i: (0, 0)),
            pl.BlockSpec((1, 256), lambda i: (0, 0)),
            pl.BlockSpec((256, 1), lambda i: (0, 0)),
            pl.BlockSpec((1, 1), lambda i: (0, 0)),
        ],
        out_specs=pl.BlockSpec((NB, 1), lambda i: (i, 0)),
        out_shape=jax.ShapeDtypeStruct((NP, 1), jnp.float32),
    )(x, acc, gamma.reshape(1, 256), beta.reshape(1, 256), W2, b2.reshape(1, 1))


def gather_rows(table, idx):
    return table[idx]


def gather_scalar(table, idx):
    return table[idx]


def scatter_add_rows(ehh, idx):
    p0 = jax.ops.segment_sum(ehh[0], idx, num_segments=NP)
    p1 = jax.ops.segment_sum(ehh[1], idx, num_segments=NP)
    z = jnp.zeros_like(p0)
    return jnp.stack([p0, z]), jnp.stack([p1, z])


def scatter_add_scalar(e, idx):
    s = jax.ops.segment_sum(e, idx, num_segments=NP)
    return jnp.stack([s, jnp.zeros_like(s)])


def gnn_edge_e(wdt, wst, dst, src):
    l = _lrelu(wdt[dst] + wst[src])
    e = jnp.exp(jnp.minimum(l, 45.0))
    return e, scatter_add_scalar(e, dst)


def kernel(node_feats, edge_feats, edge_index,
           gc_Wn, gc_bn, gc_We1, gc_be1, gc_We2, gc_be2, gc_Wet, gc_bet,
           gc_gru_Wih, gc_gru_bih, gc_gru_Whh, gc_gru_bhh,
           gnn_Wpe, gnn_bpe, gnn_Wpn, gnn_bpn,
           gnn_gru_Wih, gnn_gru_bih, gnn_gru_Whh, gnn_gru_bhh,
           pred_W1, pred_b1, pred_gamma, pred_beta, pred_W2, pred_b2):
    N, F = node_feats.shape
    E = edge_index.shape[1]
    nf_p = jnp.pad(node_feats, ((0, NP - N), (0, 0)))
    ef_p = jnp.pad(edge_feats, ((0, EP - E), (0, 0)))
    src = jnp.pad(edge_index[0], (0, EP - E), constant_values=NP - 1)
    dst = jnp.pad(edge_index[1], (0, EP - E), constant_values=NP - 1)

    hv_new, u, d1 = tc_prep(nf_p, gc_Wn, gc_bn, gc_We1[:F], gc_be1,
                            gc_We2[:G], gc_be2)

    g = gather_rows(u, src)
    dvec = gather_scalar(d1.reshape(NP), dst)
    ehh, e1 = tc_passA(g, ef_p, dvec.reshape(EP, 1), gc_We1[F:], gc_We2[G:])
    Pa, Pb = scatter_add_rows(ehh, dst)
    s = scatter_add_scalar(e1.reshape(EP), dst)
    h = tc_ctx_gru(Pa, Pb, s, hv_new, gc_Wet, gc_bet,
                   gc_gru_Wih, gc_gru_bih, gc_gru_Whh, gc_gru_bhh, True)

    L = gnn_Wpe.shape[0]
    for i in range(L):
        wd = gnn_Wpe[i][:G]
        ws = gnn_Wpe[i][G:]
        hvp, wdt, wst = tc_proj(h, gnn_Wpn[i], gnn_bpn[i], wd, ws)
        wdt = wdt.reshape(NP) + gnn_bpe[i, 0]
        e, s = gnn_edge_e(wdt, wst.reshape(NP), dst, src)
        g2 = gather_rows(hvp, src)
        ehh = tc_passB(g2, e.reshape(EP, 1))
        Pa, Pb = scatter_add_rows(ehh, dst)
        h = tc_ctx_gru(Pa, Pb, s, h, gc_Wet, gc_bet,
                       gnn_gru_Wih[i], gnn_gru_bih[i],
                       gnn_gru_Whh[i], gnn_gru_bhh[i], False)

    x, acc = tc_pred1(h, pred_W1, pred_b1)
    out = tc_pred2(x, acc, pred_gamma, pred_beta, pred_W2, pred_b2)
    return out[:N]


# speedup vs baseline: 4.5079x; 4.5079x over previous
import functools
import jax
import jax.numpy as jnp
from jax import lax
from jax.experimental import pallas as pl
from jax.experimental.pallas import tpu as pltpu
from jax.experimental.pallas import tpu_sc as plsc

NP = 10240
EP = 163840
EB = 2048
NB = 256
G = 256

_lrelu = lambda x: jnp.where(x >= 0, x, 0.01 * x)


def _elu(x):
    return jnp.where(x > 0, x, jnp.exp(jnp.minimum(x, 0.0)) - 1.0)


def _gru(x, h, Wih, bih, Whh, bhh):
    gi = jnp.dot(x, Wih, preferred_element_type=jnp.float32) + bih
    gh = jnp.dot(h, Whh, preferred_element_type=jnp.float32) + bhh
    i_r, i_z, i_n = gi[:, :G], gi[:, G:2 * G], gi[:, 2 * G:]
    h_r, h_z, h_n = gh[:, :G], gh[:, G:2 * G], gh[:, 2 * G:]
    r = jax.nn.sigmoid(i_r + h_r)
    z = jax.nn.sigmoid(i_z + h_z)
    nw = jnp.tanh(i_n + r * h_n)
    return (1.0 - z) * nw + z * h


def _k_prep(nf, Wn, bn, We1a, be1, we2d, be2, hv, u, d1):
    t1 = jnp.dot(nf[...], Wn[...], preferred_element_type=jnp.float32) + bn[...]
    hvv = _lrelu(t1)
    hv[...] = hvv
    u[...] = jnp.dot(nf[...], We1a[...], preferred_element_type=jnp.float32) + be1[...]
    d1[...] = jnp.dot(hvv, we2d[...], preferred_element_type=jnp.float32) + be2[...]


def tc_prep(nf_p, Wn, bn, We1a, be1, we2d, be2):
    n = NP // NB
    return pl.pallas_call(
        _k_prep,
        grid=(n,),
        in_specs=[
            pl.BlockSpec((NB, 256), lambda i: (i, 0)),
            pl.BlockSpec((256, 256), lambda i: (0, 0)),
            pl.BlockSpec((1, 256), lambda i: (0, 0)),
            pl.BlockSpec((256, 256), lambda i: (0, 0)),
            pl.BlockSpec((1, 256), lambda i: (0, 0)),
            pl.BlockSpec((256, 1), lambda i: (0, 0)),
            pl.BlockSpec((1, 1), lambda i: (0, 0)),
        ],
        out_specs=[
            pl.BlockSpec((NB, 256), lambda i: (i, 0)),
            pl.BlockSpec((NB, 256), lambda i: (i, 0)),
            pl.BlockSpec((NB, 1), lambda i: (i, 0)),
        ],
        out_shape=[
            jax.ShapeDtypeStruct((NP, 256), jnp.float32),
            jax.ShapeDtypeStruct((NP, 256), jnp.float32),
            jax.ShapeDtypeStruct((NP, 1), jnp.float32),
        ],
    )(nf_p, Wn, bn.reshape(1, 256), We1a, be1.reshape(1, 256), we2d, be2.reshape(1, 1))


def _k_passA(g, ef, dvec, We1b, we, ehh, e1):
    v = jnp.dot(ef[...], We1b[...], preferred_element_type=jnp.float32)
    he1 = _lrelu(g[...] + v)
    l = _lrelu(jnp.dot(he1, we[...], preferred_element_type=jnp.float32) + dvec[...])
    e = jnp.exp(jnp.minimum(l, 45.0))
    eh = e * he1
    ehh[0] = eh[:, :128]
    ehh[1] = eh[:, 128:]
    e1[...] = e


def tc_passA(g, ef_p, dvec, We1b, we):
    n = EP // EB
    return pl.pallas_call(
        _k_passA,
        grid=(n,),
        in_specs=[
            pl.BlockSpec((EB, 256), lambda i: (i, 0)),
            pl.BlockSpec((EB, 16), lambda i: (i, 0)),
            pl.BlockSpec((EB, 1), lambda i: (i, 0)),
            pl.BlockSpec((16, 256), lambda i: (0, 0)),
            pl.BlockSpec((256, 1), lambda i: (0, 0)),
        ],
        out_specs=[
            pl.BlockSpec((2, EB, 128), lambda i: (0, i, 0)),
            pl.BlockSpec((EB, 1), lambda i: (i, 0)),
        ],
        out_shape=[
            jax.ShapeDtypeStruct((2, EP, 128), jnp.float32),
            jax.ShapeDtypeStruct((EP, 1), jnp.float32),
        ],
    )(g, ef_p, dvec, We1b, we)


def _k_passB(g2, e1, ehh):
    eh = e1[...] * g2[...]
    ehh[0] = eh[:, :128]
    ehh[1] = eh[:, 128:]


def tc_passB(g2, e1):
    n = EP // EB
    return pl.pallas_call(
        _k_passB,
        grid=(n,),
        in_specs=[
            pl.BlockSpec((EB, 256), lambda i: (i, 0)),
            pl.BlockSpec((EB, 1), lambda i: (i, 0)),
        ],
        out_specs=pl.BlockSpec((2, EB, 128), lambda i: (0, i, 0)),
        out_shape=jax.ShapeDtypeStruct((2, EP, 128), jnp.float32),
    )(g2, e1)


def _k_ctx_gru(Pa, Pb, s0, s1, hv, Wet, bet, Wih, bih, Whh, bhh, hout, *, use_wet):
    P = jnp.concatenate([Pa[0] + Pa[1], Pb[0] + Pb[1]], axis=1)
    s = s0[...] + s1[...]
    Pn = P / jnp.maximum(s, 1e-30)
    if use_wet:
        c = jnp.dot(Pn, Wet[...], preferred_element_type=jnp.float32) \
            + jnp.where(s > 0, 1.0, 0.0) * bet[...]
    else:
        c = Pn
    h = _gru(_elu(c), hv[...], Wih[...], bih[...], Whh[...], bhh[...])
    hout[...] = jnp.maximum(h, 0.0)


def tc_ctx_gru(Pa, Pb, s, hv, Wet, bet, Wih, bih, Whh, bhh, use_wet):
    n = NP // NB
    s0 = s[0].reshape(NP, 1)
    s1 = s[1].reshape(NP, 1)
    return pl.pallas_call(
        functools.partial(_k_ctx_gru, use_wet=use_wet),
        grid=(n,),
        in_specs=[
            pl.BlockSpec((2, NB, 128), lambda i: (0, i, 0)),
            pl.BlockSpec((2, NB, 128), lambda i: (0, i, 0)),
            pl.BlockSpec((NB, 1), lambda i: (i, 0)),
            pl.BlockSpec((NB, 1), lambda i: (i, 0)),
            pl.BlockSpec((NB, 256), lambda i: (i, 0)),
            pl.BlockSpec((256, 256), lambda i: (0, 0)),
            pl.BlockSpec((1, 256), lambda i: (0, 0)),
            pl.BlockSpec((256, 768), lambda i: (0, 0)),
            pl.BlockSpec((1, 768), lambda i: (0, 0)),
            pl.BlockSpec((256, 768), lambda i: (0, 0)),
            pl.BlockSpec((1, 768), lambda i: (0, 0)),
        ],
        out_specs=pl.BlockSpec((NB, 256), lambda i: (i, 0)),
        out_shape=jax.ShapeDtypeStruct((NP, 256), jnp.float32),
    )(Pa, Pb, s0, s1, hv, Wet, bet.reshape(1, 256), Wih, bih.reshape(1, 768),
      Whh, bhh.reshape(1, 768))


def _k_proj(h, Wpn, bpn, wd, ws, hvp, wdt, wst):
    hh = h[...]
    hvp[...] = jnp.dot(hh, Wpn[...], preferred_element_type=jnp.float32) + bpn[...]
    wdt[...] = jnp.dot(hh, wd[...], preferred_element_type=jnp.float32)
    wst[...] = jnp.dot(hh, ws[...], preferred_element_type=jnp.float32)


def tc_proj(h, Wpn, bpn, wd, ws):
    n = NP // NB
    return pl.pallas_call(
        _k_proj,
        grid=(n,),
        in_specs=[
            pl.BlockSpec((NB, 256), lambda i: (i, 0)),
            pl.BlockSpec((256, 256), lambda i: (0, 0)),
            pl.BlockSpec((1, 256), lambda i: (0, 0)),
            pl.BlockSpec((256, 1), lambda i: (0, 0)),
            pl.BlockSpec((256, 1), lambda i: (0, 0)),
        ],
        out_specs=[
            pl.BlockSpec((NB, 256), lambda i: (i, 0)),
            pl.BlockSpec((NB, 1), lambda i: (i, 0)),
            pl.BlockSpec((NB, 1), lambda i: (i, 0)),
        ],
        out_shape=[
            jax.ShapeDtypeStruct((NP, 256), jnp.float32),
            jax.ShapeDtypeStruct((NP, 1), jnp.float32),
            jax.ShapeDtypeStruct((NP, 1), jnp.float32),
        ],
    )(h, Wpn, bpn.reshape(1, 256), wd, ws)


def _k_pred1(h, W1, b1, x, acc):
    i = pl.program_id(0)
    xv = jnp.maximum(jnp.dot(h[...], W1[...], preferred_element_type=jnp.float32)
                     + b1[...], 0.0)
    x[...] = xv
    row = i * NB + lax.broadcasted_iota(jnp.int32, (NB, 1), 0)
    m = jnp.where(row < 10000, 1.0, 0.0)
    xm = xv * m
    part = jnp.concatenate([jnp.sum(xm, axis=0, keepdims=True),
                            jnp.sum(xm * xm, axis=0, keepdims=True),
                            jnp.zeros((6, 256), jnp.float32)], axis=0)

    @pl.when(i == 0)
    def _():
        acc[...] = jnp.zeros_like(acc)
    acc[...] += part


def tc_pred1(h, W1, b1):
    n = NP // NB
    return pl.pallas_call(
        _k_pred1,
        grid=(n,),
        in_specs=[
            pl.BlockSpec((NB, 256), lambda i: (i, 0)),
            pl.BlockSpec((256, 256), lambda i: (0, 0)),
            pl.BlockSpec((1, 256), lambda i: (0, 0)),
        ],
        out_specs=[
            pl.BlockSpec((NB, 256), lambda i: (i, 0)),
            pl.BlockSpec((8, 256), lambda i: (0, 0)),
        ],
        out_shape=[
            jax.ShapeDtypeStruct((NP, 256), jnp.float32),
            jax.ShapeDtypeStruct((8, 256), jnp.float32),
        ],
    )(h, W1, b1.reshape(1, 256))


def _k_pred2(x, acc, gamma, beta, W2, b2, out):
    cnt = 10000.0
    mu = acc[0:1, :] / cnt
    var = acc[1:2, :] / cnt - mu * mu
    inv = gamma[...] / jnp.sqrt(var + 1e-5)
    xn = (x[...] - mu) * inv + beta[...]
    out[...] = jnp.dot(xn, W2[...], preferred_element_type=jnp.float32) + b2[...]


def tc_pred2(x, acc, gamma, beta, W2, b2):
    n = NP // NB
    return pl.pallas_call(
        _k_pred2,
        grid=(n,),
        in_specs=[
            pl.BlockSpec((NB, 256), lambda i: (i, 0)),
            pl.BlockSpec((8, 256), lambda i: (0, 0)),
            pl.BlockSpec((1, 256), lambda i: (0, 0)),
            pl.BlockSpec((1, 256), lambda i: (0, 0)),
            pl.BlockSpec((256, 1), lambda i: (0, 0)),
            pl.BlockSpec((1, 1), lambda i: (0, 0)),
        ],
        out_specs=pl.BlockSpec((NB, 1), lambda i: (i, 0)),
        out_shape=jax.ShapeDtypeStruct((NP, 1), jnp.float32),
    )(x, acc, gamma.reshape(1, 256), beta.reshape(1, 256), W2, b2.reshape(1, 1))


NC, NS = 2, 16
NW = NC * NS
RPW = EP // NW
CH = 256
NCH = RPW // CH
NPS = NP // NS

_sc_mesh = plsc.VectorSubcoreMesh(core_axis_name="c", subcore_axis_name="s")


def _wid():
    return lax.axis_index("s") * NC + lax.axis_index("c")


@functools.partial(
    pl.kernel,
    out_type=jax.ShapeDtypeStruct((EP, 256), jnp.float32),
    mesh=_sc_mesh,
    compiler_params=pltpu.CompilerParams(needs_layout_passes=False),
    scratch_types=[
        pltpu.VMEM((CH,), jnp.int32),
        pltpu.VMEM((CH, 256), jnp.float32),
        pltpu.SemaphoreType.DMA,
    ],
)
def _sc_gather_rows(table_hbm, idx_hbm, out_hbm, idx_v, rows_v, sem):
    base = _wid() * RPW

    def body(j, carry):
        off = base + j * CH
        pltpu.sync_copy(idx_hbm.at[pl.ds(off, CH)], idx_v)
        pltpu.async_copy(table_hbm.at[idx_v], rows_v, sem).wait()
        pltpu.sync_copy(rows_v, out_hbm.at[pl.ds(off, CH)])
        return carry

    lax.fori_loop(0, NCH, body, 0)


def gather_rows(table, idx):
    return _sc_gather_rows(table, idx)


@functools.partial(
    pl.kernel,
    out_type=jax.ShapeDtypeStruct((EP,), jnp.float32),
    mesh=_sc_mesh,
    compiler_params=pltpu.CompilerParams(needs_layout_passes=False),
    scratch_types=[
        pltpu.VMEM((NP,), jnp.float32),
        pltpu.VMEM((CH,), jnp.int32),
        pltpu.VMEM((CH,), jnp.float32),
        pltpu.SemaphoreType.DMA,
    ],
)
def _sc_gather_scalar(table_hbm, idx_hbm, out_hbm, tab_v, idx_v, val_v, sem):
    pltpu.sync_copy(table_hbm, tab_v)
    base = _wid() * RPW

    def body(j, carry):
        off = base + j * CH
        pltpu.sync_copy(idx_hbm.at[pl.ds(off, CH)], idx_v)

        def inner(k, c2):
            idx16 = idx_v[pl.ds(k * 16, 16)]
            val_v[pl.ds(k * 16, 16)] = plsc.load_gather(tab_v, [idx16])
            return c2

        lax.fori_loop(0, CH // 16, inner, 0)
        pltpu.sync_copy(val_v, out_hbm.at[pl.ds(off, CH)])
        return carry

    lax.fori_loop(0, NCH, body, 0)


def gather_scalar(table, idx):
    return _sc_gather_scalar(table, idx)


def _make_sc_scatter(D):
    if D == 1:
        out_t = jax.ShapeDtypeStruct((NC, NP), jnp.float32)
        data_t = pltpu.VMEM((CH,), jnp.float32)
        acc_t = pltpu.VMEM_SHARED((NP,), jnp.float32)
    else:
        out_t = jax.ShapeDtypeStruct((NC, NP, D), jnp.float32)
        data_t = pltpu.VMEM((CH, D), jnp.float32)
        acc_t = pltpu.VMEM_SHARED((NP, D), jnp.float32)

    @functools.partial(
        pl.kernel,
        out_type=out_t,
        mesh=_sc_mesh,
        compiler_params=pltpu.CompilerParams(needs_layout_passes=False),
        scratch_types=[
            pltpu.VMEM((CH,), jnp.int32),
            data_t,
            acc_t,
            pltpu.SemaphoreType.DMA,
        ],
    )
    def k(data_hbm, idx_hbm, zeros_hbm, out_hbm, idx_v, rows_v, acc, sem):
        cid = lax.axis_index("c")
        sid = lax.axis_index("s")
        pltpu.sync_copy(zeros_hbm, acc.at[pl.ds(sid * NPS, NPS)])
        plsc.subcore_barrier()
        base = (sid * NC + cid) * RPW

        def body(j, carry):
            off = base + j * CH
            pltpu.sync_copy(idx_hbm.at[pl.ds(off, CH)], idx_v)
            pltpu.sync_copy(data_hbm.at[pl.ds(off, CH)], rows_v)
            pltpu.sync_copy(rows_v, acc.at[idx_v], add=True)
            return carry

        lax.fori_loop(0, NCH, body, 0)
        plsc.subcore_barrier()
        pltpu.sync_copy(acc.at[pl.ds(sid * NPS, NPS)],
                        out_hbm.at[cid, pl.ds(sid * NPS, NPS)])

    return k


_sc_scatter_rows = _make_sc_scatter(128)
_sc_scatter_scalar = _make_sc_scatter(1)


def scatter_add_rows(ehh, idx):
    z = jnp.zeros((NPS, 128), jnp.float32)
    return (_sc_scatter_rows(ehh[0], idx, z),
            _sc_scatter_rows(ehh[1], idx, z))


def scatter_add_scalar(e, idx):
    z = jnp.zeros((NPS,), jnp.float32)
    return _sc_scatter_scalar(e, idx, z)


@functools.partial(
    pl.kernel,
    out_type=[jax.ShapeDtypeStruct((EP,), jnp.float32),
              jax.ShapeDtypeStruct((NC, NP), jnp.float32)],
    mesh=_sc_mesh,
    compiler_params=pltpu.CompilerParams(needs_layout_passes=False),
    scratch_types=[
        pltpu.VMEM((NP,), jnp.float32),
        pltpu.VMEM((NP,), jnp.float32),
        pltpu.VMEM((CH,), jnp.int32),
        pltpu.VMEM((CH,), jnp.int32),
        pltpu.VMEM((CH,), jnp.float32),
        pltpu.VMEM_SHARED((NP,), jnp.float32),
        pltpu.SemaphoreType.DMA,
    ],
)
def _sc_gnn_edge(wd_hbm, ws_hbm, dst_hbm, src_hbm, zeros_hbm, e_hbm, s_hbm,
                 wd_v, ws_v, idxd_v, idxs_v, e_v, acc, sem):
    cid = lax.axis_index("c")
    sid = lax.axis_index("s")
    pltpu.sync_copy(wd_hbm, wd_v)
    pltpu.sync_copy(ws_hbm, ws_v)
    pltpu.sync_copy(zeros_hbm, acc.at[pl.ds(sid * NPS, NPS)])
    plsc.subcore_barrier()
    base = (sid * NC + cid) * RPW

    def body(j, carry):
        off = base + j * CH
        pltpu.sync_copy(dst_hbm.at[pl.ds(off, CH)], idxd_v)
        pltpu.sync_copy(src_hbm.at[pl.ds(off, CH)], idxs_v)

        def inner(k, c2):
            a = plsc.load_gather(wd_v, [idxd_v[pl.ds(k * 16, 16)]])
            b = plsc.load_gather(ws_v, [idxs_v[pl.ds(k * 16, 16)]])
            l = a + b
            l = jnp.where(l >= 0.0, l, 0.01 * l)
            e_v[pl.ds(k * 16, 16)] = jnp.exp(jnp.minimum(l, 45.0))
            return c2

        lax.fori_loop(0, CH // 16, inner, 0)
        pltpu.sync_copy(e_v, e_hbm.at[pl.ds(off, CH)])
        pltpu.sync_copy(e_v, acc.at[idxd_v], add=True)
        return carry

    lax.fori_loop(0, NCH, body, 0)
    plsc.subcore_barrier()
    pltpu.sync_copy(acc.at[pl.ds(sid * NPS, NPS)],
                    s_hbm.at[cid, pl.ds(sid * NPS, NPS)])


def gnn_edge_e(wdt, wst, dst, src):
    z = jnp.zeros((NPS,), jnp.float32)
    return _sc_gnn_edge(wdt, wst, dst, src, z)


def kernel(node_feats, edge_feats, edge_index,
           gc_Wn, gc_bn, gc_We1, gc_be1, gc_We2, gc_be2, gc_Wet, gc_bet,
           gc_gru_Wih, gc_gru_bih, gc_gru_Whh, gc_gru_bhh,
           gnn_Wpe, gnn_bpe, gnn_Wpn, gnn_bpn,
           gnn_gru_Wih, gnn_gru_bih, gnn_gru_Whh, gnn_gru_bhh,
           pred_W1, pred_b1, pred_gamma, pred_beta, pred_W2, pred_b2):
    N, F = node_feats.shape
    E = edge_index.shape[1]
    nf_p = jnp.pad(node_feats, ((0, NP - N), (0, 0)))
    ef_p = jnp.pad(edge_feats, ((0, EP - E), (0, 0)))
    src = jnp.pad(edge_index[0], (0, EP - E), constant_values=NP - 1)
    dst = jnp.pad(edge_index[1], (0, EP - E), constant_values=NP - 1)

    hv_new, u, d1 = tc_prep(nf_p, gc_Wn, gc_bn, gc_We1[:F], gc_be1,
                            gc_We2[:G], gc_be2)

    g = gather_rows(u, src)
    dvec = gather_scalar(d1.reshape(NP), dst)
    ehh, e1 = tc_passA(g, ef_p, dvec.reshape(EP, 1), gc_We1[F:], gc_We2[G:])
    Pa, Pb = scatter_add_rows(ehh, dst)
    s = scatter_add_scalar(e1.reshape(EP), dst)
    h = tc_ctx_gru(Pa, Pb, s, hv_new, gc_Wet, gc_bet,
                   gc_gru_Wih, gc_gru_bih, gc_gru_Whh, gc_gru_bhh, True)

    L = gnn_Wpe.shape[0]
    for i in range(L):
        wd = gnn_Wpe[i][:G]
        ws = gnn_Wpe[i][G:]
        hvp, wdt, wst = tc_proj(h, gnn_Wpn[i], gnn_bpn[i], wd, ws)
        wdt = wdt.reshape(NP) + gnn_bpe[i, 0]
        e, s = gnn_edge_e(wdt, wst.reshape(NP), dst, src)
        g2 = gather_rows(hvp, src)
        ehh = tc_passB(g2, e.reshape(EP, 1))
        Pa, Pb = scatter_add_rows(ehh, dst)
        h = tc_ctx_gru(Pa, Pb, s, h, gc_Wet, gc_bet,
                       gnn_gru_Wih[i], gnn_gru_bih[i],
                       gnn_gru_Whh[i], gnn_gru_bhh[i], False)

    x, acc = tc_pred1(h, pred_W1, pred_b1)
    out = tc_pred2(x, acc, pred_gamma, pred_beta, pred_W2, pred_b2)
    return out[:N]


# speedup vs baseline: 6.3851x; 1.4164x over previous
import functools
import jax
import jax.numpy as jnp
from jax import lax
from jax.experimental import pallas as pl
from jax.experimental.pallas import tpu as pltpu
from jax.experimental.pallas import tpu_sc as plsc

NP = 10240
EP = 163840
EB = 2048
NB = 256
G = 256

_lrelu = lambda x: jnp.where(x >= 0, x, 0.01 * x)


def _elu(x):
    return jnp.where(x > 0, x, jnp.exp(jnp.minimum(x, 0.0)) - 1.0)


def _gru(x, h, Wih, bih, Whh, bhh):
    gi = jnp.dot(x, Wih, preferred_element_type=jnp.float32) + bih
    gh = jnp.dot(h, Whh, preferred_element_type=jnp.float32) + bhh
    i_r, i_z, i_n = gi[:, :G], gi[:, G:2 * G], gi[:, 2 * G:]
    h_r, h_z, h_n = gh[:, :G], gh[:, G:2 * G], gh[:, 2 * G:]
    r = jax.nn.sigmoid(i_r + h_r)
    z = jax.nn.sigmoid(i_z + h_z)
    nw = jnp.tanh(i_n + r * h_n)
    return (1.0 - z) * nw + z * h


def _k_prep(nf, Wn, bn, We1a, be1, we2d, be2, hv, u0, u1, d1):
    t1 = jnp.dot(nf[...], Wn[...], preferred_element_type=jnp.float32) + bn[...]
    hvv = _lrelu(t1)
    hv[...] = hvv
    uu = jnp.dot(nf[...], We1a[...], preferred_element_type=jnp.float32) + be1[...]
    u0[...] = uu[:, :128]
    u1[...] = uu[:, 128:]
    d1[...] = jnp.dot(hvv, we2d[...], preferred_element_type=jnp.float32) + be2[...]


def tc_prep(nf_p, Wn, bn, We1a, be1, we2d, be2):
    n = NP // NB
    return pl.pallas_call(
        _k_prep,
        grid=(n,),
        in_specs=[
            pl.BlockSpec((NB, 256), lambda i: (i, 0)),
            pl.BlockSpec((256, 256), lambda i: (0, 0)),
            pl.BlockSpec((1, 256), lambda i: (0, 0)),
            pl.BlockSpec((256, 256), lambda i: (0, 0)),
            pl.BlockSpec((1, 256), lambda i: (0, 0)),
            pl.BlockSpec((256, 1), lambda i: (0, 0)),
            pl.BlockSpec((1, 1), lambda i: (0, 0)),
        ],
        out_specs=[
            pl.BlockSpec((NB, 256), lambda i: (i, 0)),
            pl.BlockSpec((NB, 128), lambda i: (i, 0)),
            pl.BlockSpec((NB, 128), lambda i: (i, 0)),
            pl.BlockSpec((NB, 1), lambda i: (i, 0)),
        ],
        out_shape=[
            jax.ShapeDtypeStruct((NP, 256), jnp.float32),
            jax.ShapeDtypeStruct((NP, 128), jnp.float32),
            jax.ShapeDtypeStruct((NP, 128), jnp.float32),
            jax.ShapeDtypeStruct((NP, 1), jnp.float32),
        ],
    )(nf_p, Wn, bn.reshape(1, 256), We1a, be1.reshape(1, 256), we2d, be2.reshape(1, 1))


def _k_passA(g, ef, dvec, We1b, we, eh0, eh1, e1):
    v = jnp.dot(ef[...], We1b[...], preferred_element_type=jnp.float32)
    he1 = _lrelu(g[...] + v)
    l = _lrelu(jnp.dot(he1, we[...], preferred_element_type=jnp.float32) + dvec[...])
    e = jnp.exp(jnp.minimum(l, 45.0))
    eh = e * he1
    eh0[...] = eh[:, :128]
    eh1[...] = eh[:, 128:]
    e1[...] = e


def tc_passA(g, ef_p, dvec, We1b, we):
    n = EP // EB
    return pl.pallas_call(
        _k_passA,
        grid=(n,),
        in_specs=[
            pl.BlockSpec((EB, 256), lambda i: (i, 0)),
            pl.BlockSpec((EB, 16), lambda i: (i, 0)),
            pl.BlockSpec((EB, 1), lambda i: (i, 0)),
            pl.BlockSpec((16, 256), lambda i: (0, 0)),
            pl.BlockSpec((256, 1), lambda i: (0, 0)),
        ],
        out_specs=[
            pl.BlockSpec((EB, 128), lambda i: (i, 0)),
            pl.BlockSpec((EB, 128), lambda i: (i, 0)),
            pl.BlockSpec((EB, 1), lambda i: (i, 0)),
        ],
        out_shape=[
            jax.ShapeDtypeStruct((EP, 128), jnp.float32),
            jax.ShapeDtypeStruct((EP, 128), jnp.float32),
            jax.ShapeDtypeStruct((EP, 1), jnp.float32),
        ],
    )(g, ef_p, dvec, We1b, we)


def _k_ctx_gru(P, s1, hv, Wet, bet, Wih, bih, Whh, bhh, hout, *, use_wet):
    s = s1[...]
    Pn = P[...] / jnp.maximum(s, 1e-30)
    if use_wet:
        c = jnp.dot(Pn, Wet[...], preferred_element_type=jnp.float32) \
            + jnp.where(s > 0, 1.0, 0.0) * bet[...]
    else:
        c = Pn
    h = _gru(_elu(c), hv[...], Wih[...], bih[...], Whh[...], bhh[...])
    hout[...] = jnp.maximum(h, 0.0)


def tc_ctx_gru(P, s, hv, Wet, bet, Wih, bih, Whh, bhh, use_wet):
    n = NP // NB
    return pl.pallas_call(
        functools.partial(_k_ctx_gru, use_wet=use_wet),
        grid=(n,),
        in_specs=[
            pl.BlockSpec((NB, 256), lambda i: (i, 0)),
            pl.BlockSpec((NB, 1), lambda i: (i, 0)),
            pl.BlockSpec((NB, 256), lambda i: (i, 0)),
            pl.BlockSpec((256, 256), lambda i: (0, 0)),
            pl.BlockSpec((1, 256), lambda i: (0, 0)),
            pl.BlockSpec((256, 768), lambda i: (0, 0)),
            pl.BlockSpec((1, 768), lambda i: (0, 0)),
            pl.BlockSpec((256, 768), lambda i: (0, 0)),
            pl.BlockSpec((1, 768), lambda i: (0, 0)),
        ],
        out_specs=pl.BlockSpec((NB, 256), lambda i: (i, 0)),
        out_shape=jax.ShapeDtypeStruct((NP, 256), jnp.float32),
    )(P, s.reshape(NP, 1), hv, Wet, bet.reshape(1, 256), Wih,
      bih.reshape(1, 768), Whh, bhh.reshape(1, 768))


def _k_proj(h, Wpn, bpn, wd, ws, bpe, hvp0, hvp1, wdt, wst):
    hh = h[...]
    pv = jnp.dot(hh, Wpn[...], preferred_element_type=jnp.float32) + bpn[...]
    hvp0[...] = pv[:, :128]
    hvp1[...] = pv[:, 128:]
    wdt[...] = jnp.dot(hh, wd[...], preferred_element_type=jnp.float32) + bpe[...]
    wst[...] = jnp.dot(hh, ws[...], preferred_element_type=jnp.float32)


def tc_proj(h, Wpn, bpn, wd, ws, bpe):
    n = NP // NB
    return pl.pallas_call(
        _k_proj,
        grid=(n,),
        in_specs=[
            pl.BlockSpec((NB, 256), lambda i: (i, 0)),
            pl.BlockSpec((256, 256), lambda i: (0, 0)),
            pl.BlockSpec((1, 256), lambda i: (0, 0)),
            pl.BlockSpec((256, 1), lambda i: (0, 0)),
            pl.BlockSpec((256, 1), lambda i: (0, 0)),
            pl.BlockSpec((1, 1), lambda i: (0, 0)),
        ],
        out_specs=[
            pl.BlockSpec((NB, 128), lambda i: (i, 0)),
            pl.BlockSpec((NB, 128), lambda i: (i, 0)),
            pl.BlockSpec((NB, 1), lambda i: (i, 0)),
            pl.BlockSpec((NB, 1), lambda i: (i, 0)),
        ],
        out_shape=[
            jax.ShapeDtypeStruct((NP, 128), jnp.float32),
            jax.ShapeDtypeStruct((NP, 128), jnp.float32),
            jax.ShapeDtypeStruct((NP, 1), jnp.float32),
            jax.ShapeDtypeStruct((NP, 1), jnp.float32),
        ],
    )(h, Wpn, bpn.reshape(1, 256), wd, ws, bpe.reshape(1, 1))


def _k_pred1(h, W1, b1, x, acc):
    i = pl.program_id(0)
    xv = jnp.maximum(jnp.dot(h[...], W1[...], preferred_element_type=jnp.float32)
                     + b1[...], 0.0)
    x[...] = xv
    row = i * NB + lax.broadcasted_iota(jnp.int32, (NB, 1), 0)
    m = jnp.where(row < 10000, 1.0, 0.0)
    xm = xv * m
    part = jnp.concatenate([jnp.sum(xm, axis=0, keepdims=True),
                            jnp.sum(xm * xm, axis=0, keepdims=True),
                            jnp.zeros((6, 256), jnp.float32)], axis=0)

    @pl.when(i == 0)
    def _():
        acc[...] = jnp.zeros_like(acc)
    acc[...] += part


def tc_pred1(h, W1, b1):
    n = NP // NB
    return pl.pallas_call(
        _k_pred1,
        grid=(n,),
        in_specs=[
            pl.BlockSpec((NB, 256), lambda i: (i, 0)),
            pl.BlockSpec((256, 256), lambda i: (0, 0)),
            pl.BlockSpec((1, 256), lambda i: (0, 0)),
        ],
        out_specs=[
            pl.BlockSpec((NB, 256), lambda i: (i, 0)),
            pl.BlockSpec((8, 256), lambda i: (0, 0)),
        ],
        out_shape=[
            jax.ShapeDtypeStruct((NP, 256), jnp.float32),
            jax.ShapeDtypeStruct((8, 256), jnp.float32),
        ],
    )(h, W1, b1.reshape(1, 256))


def _k_pred2(x, acc, gamma, beta, W2, b2, out):
    cnt = 10000.0
    mu = acc[0:1, :] / cnt
    var = acc[1:2, :] / cnt - mu * mu
    inv = gamma[...] / jnp.sqrt(var + 1e-5)
    xn = (x[...] - mu) * inv + beta[...]
    out[...] = jnp.dot(xn, W2[...], preferred_element_type=jnp.float32) + b2[...]


def tc_pred2(x, acc, gamma, beta, W2, b2):
    n = NP // NB
    return pl.pallas_call(
        _k_pred2,
        grid=(n,),
        in_specs=[
            pl.BlockSpec((NB, 256), lambda i: (i, 0)),
            pl.BlockSpec((8, 256), lambda i: (0, 0)),
            pl.BlockSpec((1, 256), lambda i: (0, 0)),
            pl.BlockSpec((1, 256), lambda i: (0, 0)),
            pl.BlockSpec((256, 1), lambda i: (0, 0)),
            pl.BlockSpec((1, 1), lambda i: (0, 0)),
        ],
        out_specs=pl.BlockSpec((NB, 1), lambda i: (i, 0)),
        out_shape=jax.ShapeDtypeStruct((NP, 1), jnp.float32),
    )(x, acc, gamma.reshape(1, 256), beta.reshape(1, 256), W2, b2.reshape(1, 1))


NC, NS = 2, 16
RPW = EP // NS
CH = 256
NCH = RPW // CH
NPS = NP // NS

_sc_mesh = plsc.VectorSubcoreMesh(core_axis_name="c", subcore_axis_name="s")
_sc_params = pltpu.CompilerParams(needs_layout_passes=False)


def _sc_e16(wd_v, ws_v, idxd_v, idxs_v, k):
    a = plsc.load_gather(wd_v, [idxd_v[pl.ds(k * 16, 16)]])
    b = plsc.load_gather(ws_v, [idxs_v[pl.ds(k * 16, 16)]])
    l = a + b
    l = jnp.where(l >= 0.0, l, 0.01 * l)
    return jnp.exp(jnp.minimum(l, 45.0))


@functools.partial(
    pl.kernel,
    out_type=[jax.ShapeDtypeStruct((EP, 256), jnp.float32),
              jax.ShapeDtypeStruct((EP,), jnp.float32)],
    mesh=_sc_mesh,
    compiler_params=_sc_params,
    scratch_types=[
        pltpu.VMEM((NP,), jnp.float32),
        pltpu.VMEM((CH,), jnp.int32),
        pltpu.VMEM((CH,), jnp.int32),
        pltpu.VMEM((CH,), jnp.float32),
        pltpu.VMEM((CH, 128), jnp.float32),
        pltpu.SemaphoreType.DMA,
    ],
)
def _sc_gather_ctx(u0_hbm, u1_hbm, dt_hbm, src_hbm, dst_hbm, g_hbm, dvec_hbm,
                   dt_v, idxs_v, idxd_v, val_v, rows_v, sem):
    cid = lax.axis_index("c")
    sid = lax.axis_index("s")
    base = sid * RPW
    pltpu.sync_copy(dt_hbm, dt_v)

    def body(j, carry):
        off = base + j * CH
        pltpu.sync_copy(src_hbm.at[pl.ds(off, CH)], idxs_v)

        @pl.when(cid == 0)
        def _():
            pltpu.async_copy(u0_hbm.at[idxs_v], rows_v, sem)

        @pl.when(cid == 1)
        def _():
            pltpu.async_copy(u1_hbm.at[idxs_v], rows_v, sem)

        @pl.when(cid == 0)
        def _():
            pltpu.sync_copy(dst_hbm.at[pl.ds(off, CH)], idxd_v)

            def inner(k, c2):
                val_v[pl.ds(k * 16, 16)] = plsc.load_gather(
                    dt_v, [idxd_v[pl.ds(k * 16, 16)]])
                return c2

            lax.fori_loop(0, CH // 16, inner, 0)

        pltpu.make_async_copy(u0_hbm.at[idxs_v], rows_v, sem).wait()
        pltpu.sync_copy(rows_v,
                        g_hbm.at[pl.ds(off, CH), pl.ds(cid * 128, 128)])

        @pl.when(cid == 0)
        def _():
            pltpu.sync_copy(val_v, dvec_hbm.at[pl.ds(off, CH)])

        return carry

    lax.fori_loop(0, NCH, body, 0)


@functools.partial(
    pl.kernel,
    out_type=[jax.ShapeDtypeStruct((NP, 256), jnp.float32),
              jax.ShapeDtypeStruct((NP,), jnp.float32)],
    mesh=_sc_mesh,
    compiler_params=_sc_params,
    scratch_types=[
        pltpu.VMEM((CH,), jnp.int32),
        pltpu.VMEM((CH,), jnp.float32),
        pltpu.VMEM((CH, 128), jnp.float32),
        pltpu.VMEM_SHARED((NP, 128), jnp.float32),
        pltpu.VMEM_SHARED((NP,), jnp.float32),
        pltpu.SemaphoreType.DMA,
    ],
)
def _sc_scatter_ctx(eh0_hbm, eh1_hbm, e_hbm, dst_hbm, z128_hbm, z1_hbm,
                    p_hbm, s_hbm,
                    idx_v, e_v, rows_v, acc, accs, sem):
    cid = lax.axis_index("c")
    sid = lax.axis_index("s")
    base = sid * RPW
    pltpu.sync_copy(z128_hbm, acc.at[pl.ds(sid * NPS, NPS)])

    @pl.when(cid == 0)
    def _():
        pltpu.sync_copy(z1_hbm, accs.at[pl.ds(sid * NPS, NPS)])

    plsc.subcore_barrier()

    def body(j, carry):
        off = base + j * CH
        pltpu.sync_copy(dst_hbm.at[pl.ds(off, CH)], idx_v)

        @pl.when(cid == 0)
        def _():
            pltpu.sync_copy(eh0_hbm.at[pl.ds(off, CH)], rows_v)

        @pl.when(cid == 1)
        def _():
            pltpu.sync_copy(eh1_hbm.at[pl.ds(off, CH)], rows_v)

        pltpu.sync_copy(rows_v, acc.at[idx_v], add=True)

        @pl.when(cid == 0)
        def _():
            pltpu.sync_copy(e_hbm.at[pl.ds(off, CH)], e_v)
            pltpu.sync_copy(e_v, accs.at[idx_v], add=True)

        return carry

    lax.fori_loop(0, NCH, body, 0)
    plsc.subcore_barrier()
    pltpu.sync_copy(acc.at[pl.ds(sid * NPS, NPS)],
                    p_hbm.at[pl.ds(sid * NPS, NPS), pl.ds(cid * 128, 128)])

    @pl.when(cid == 0)
    def _():
        pltpu.sync_copy(accs.at[pl.ds(sid * NPS, NPS)],
                        s_hbm.at[pl.ds(sid * NPS, NPS)])


CHG = 160
NCHG = RPW // CHG


@functools.partial(
    pl.kernel,
    out_type=[jax.ShapeDtypeStruct((NP, 256), jnp.float32),
              jax.ShapeDtypeStruct((NP,), jnp.float32)],
    mesh=_sc_mesh,
    compiler_params=_sc_params,
    scratch_types=[
        pltpu.VMEM((NP,), jnp.float32),
        pltpu.VMEM((NP,), jnp.float32),
        pltpu.VMEM((CHG,), jnp.int32),
        pltpu.VMEM((CHG,), jnp.int32),
        pltpu.VMEM((CHG + 16,), jnp.float32),
        pltpu.VMEM((CHG, 128), jnp.float32),
        pltpu.VMEM_SHARED((NP, 128), jnp.float32),
        pltpu.VMEM_SHARED((NP,), jnp.float32),
        pltpu.SemaphoreType.DMA,
    ],
)
def _sc_gnn_layer(hvp0_hbm, hvp1_hbm, wd_hbm, ws_hbm, dst_hbm, src_hbm,
                  z128_hbm, z1_hbm, p_hbm, s_hbm,
                  wd_v, ws_v, idxd_v, idxs_v, e_v, rows_v, acc, accs, sem):
    cid = lax.axis_index("c")
    sid = lax.axis_index("s")
    base = sid * RPW
    pltpu.sync_copy(wd_hbm, wd_v)
    pltpu.sync_copy(ws_hbm, ws_v)
    pltpu.sync_copy(z128_hbm, acc.at[pl.ds(sid * NPS, NPS)])

    @pl.when(cid == 0)
    def _():
        pltpu.sync_copy(z1_hbm, accs.at[pl.ds(sid * NPS, NPS)])

    plsc.subcore_barrier()

    def body(j, carry):
        off = base + j * CHG
        pltpu.sync_copy(dst_hbm.at[pl.ds(off, CHG)], idxd_v)
        pltpu.sync_copy(src_hbm.at[pl.ds(off, CHG)], idxs_v)

        @pl.when(cid == 0)
        def _():
            pltpu.async_copy(hvp0_hbm.at[idxs_v], rows_v, sem)

        @pl.when(cid == 1)
        def _():
            pltpu.async_copy(hvp1_hbm.at[idxs_v], rows_v, sem)

        def inner(k, c2):
            e_v[pl.ds(k * 16, 16)] = _sc_e16(wd_v, ws_v, idxd_v, idxs_v, k)
            return c2

        lax.fori_loop(0, CHG // 16, inner, 0)
        pltpu.make_async_copy(hvp0_hbm.at[idxs_v], rows_v, sem).wait()

        def row(i, c):
            ev = e_v[pl.ds(i, 16)][0]
            for kk in range(8):
                sl = pl.ds(kk * 16, 16)
                rows_v[i, sl] = rows_v[i, sl] * ev
            return c

        lax.fori_loop(0, CHG, row, 0)
        pltpu.sync_copy(rows_v, acc.at[idxd_v], add=True)

        @pl.when(cid == 0)
        def _():
            pltpu.sync_copy(e_v.at[pl.ds(0, CHG)], accs.at[idxd_v], add=True)

        return carry

    lax.fori_loop(0, NCHG, body, 0)
    plsc.subcore_barrier()
    pltpu.sync_copy(acc.at[pl.ds(sid * NPS, NPS)],
                    p_hbm.at[pl.ds(sid * NPS, NPS), pl.ds(cid * 128, 128)])

    @pl.when(cid == 0)
    def _():
        pltpu.sync_copy(accs.at[pl.ds(sid * NPS, NPS)],
                        s_hbm.at[pl.ds(sid * NPS, NPS)])


def kernel(node_feats, edge_feats, edge_index,
           gc_Wn, gc_bn, gc_We1, gc_be1, gc_We2, gc_be2, gc_Wet, gc_bet,
           gc_gru_Wih, gc_gru_bih, gc_gru_Whh, gc_gru_bhh,
           gnn_Wpe, gnn_bpe, gnn_Wpn, gnn_bpn,
           gnn_gru_Wih, gnn_gru_bih, gnn_gru_Whh, gnn_gru_bhh,
           pred_W1, pred_b1, pred_gamma, pred_beta, pred_W2, pred_b2):
    N, F = node_feats.shape
    E = edge_index.shape[1]
    nf_p = jnp.pad(node_feats, ((0, NP - N), (0, 0)))
    ef_p = jnp.pad(edge_feats, ((0, EP - E), (0, 0)))
    src = jnp.pad(edge_index[0], (0, EP - E), constant_values=NP - 1)
    dst = jnp.pad(edge_index[1], (0, EP - E), constant_values=NP - 1)

    hv_new, u0, u1, d1 = tc_prep(nf_p, gc_Wn, gc_bn, gc_We1[:F], gc_be1,
                                 gc_We2[:G], gc_be2)
    z128 = jnp.zeros((NPS, 128), jnp.float32)
    z1 = jnp.zeros((NPS,), jnp.float32)

    g, dvec = _sc_gather_ctx(u0, u1, d1.reshape(NP), src, dst)
    eh0, eh1, e1 = tc_passA(g, ef_p, dvec.reshape(EP, 1), gc_We1[F:],
                            gc_We2[G:])
    P, s = _sc_scatter_ctx(eh0, eh1, e1.reshape(EP), dst, z128, z1)
    h = tc_ctx_gru(P, s, hv_new, gc_Wet, gc_bet,
                   gc_gru_Wih, gc_gru_bih, gc_gru_Whh, gc_gru_bhh, True)

    L = gnn_Wpe.shape[0]
    for i in range(L):
        hvp0, hvp1, wdt, wst = tc_proj(h, gnn_Wpn[i], gnn_bpn[i],
                                       gnn_Wpe[i][:G], gnn_Wpe[i][G:],
                                       gnn_bpe[i])
        P, s = _sc_gnn_layer(hvp0, hvp1, wdt.reshape(NP), wst.reshape(NP),
                             dst, src, z128, z1)
        h = tc_ctx_gru(P, s, h, gc_Wet, gc_bet,
                       gnn_gru_Wih[i], gnn_gru_bih[i],
                       gnn_gru_Whh[i], gnn_gru_bhh[i], False)

    x, acc = tc_pred1(h, pred_W1, pred_b1)
    out = tc_pred2(x, acc, pred_gamma, pred_beta, pred_W2, pred_b2)
    return out[:N]
